# baseline (speedup 1.0000x reference)
import functools

import jax
import jax.numpy as jnp
from jax import lax
from jax.experimental import pallas as pl
from jax.experimental.pallas import tpu as pltpu
from jax.experimental.pallas import tpu_sc as plsc

N_NODES = 100000
B = 2048
D = 128
HIDDEN = 64
OUT_DIM = 2

NC = 2
NS = 16
NW = NC * NS
CHUNK = 400
NCHUNKS = N_NODES // CHUNK
BASE_PER_W = NCHUNKS // NW
EXTRA = NCHUNKS - BASE_PER_W * NW
MAXC = BASE_PER_W + 1
NBUF = 2
SEG_PER_TILE = B // NS


def _sc_segment_sum(x, idx):
    mesh = plsc.VectorSubcoreMesh(core_axis_name="c", subcore_axis_name="s")

    @functools.partial(
        pl.kernel,
        mesh=mesh,
        out_type=jax.ShapeDtypeStruct((NC, B, D), jnp.float32),
        scratch_types=[
            pltpu.VMEM((NBUF, CHUNK, D), jnp.float32),
            pltpu.VMEM((CHUNK,), jnp.int32),
            pltpu.VMEM((CHUNK,), jnp.int32),
            pltpu.VMEM((CHUNK,), jnp.int32),
            pltpu.VMEM((CHUNK,), jnp.int32),
            pltpu.VMEM_SHARED((B, D), jnp.float32),
            pltpu.SemaphoreType.DMA,
            pltpu.SemaphoreType.DMA,
            pltpu.SemaphoreType.DMA,
            pltpu.SemaphoreType.DMA,
            pltpu.SemaphoreType.DMA,
            pltpu.SemaphoreType.DMA,
            pltpu.SemaphoreType.DMA,
            pltpu.SemaphoreType.DMA,
        ],
    )
    def seg_sum(x_hbm, idx_hbm, out_hbm, rows_v, idx_v0, idx_v1, idx_v2,
                idx_v3, acc_sh, g0, g1, g2, g3, s0, s1, s2, s3):
        c = lax.axis_index("c")
        s = lax.axis_index("s")
        wid = s * NC + c
        gsems = (g0, g1, g2, g3)
        ssems = (s0, s1, s2, s3)
        idx_bufs = (idx_v0, idx_v1, idx_v2, idx_v3)

        zvec = jnp.zeros((16,), jnp.float32)

        def zero_row(r, _):
            for j in range(D // 16):
                rows_v[0, r, pl.ds(j * 16, 16)] = zvec
            return 0

        lax.fori_loop(0, SEG_PER_TILE, zero_row, 0)
        pltpu.sync_copy(rows_v.at[0, pl.ds(0, SEG_PER_TILE)],
                        acc_sh.at[pl.ds(s * SEG_PER_TILE, SEG_PER_TILE)])
        plsc.subcore_barrier()

        first = BASE_PER_W * wid + jnp.minimum(wid, EXTRA)
        count = jnp.where(wid < EXTRA, BASE_PER_W + 1, BASE_PER_W)

        def start_g(j, b):
            r0 = (first + j) * CHUNK
            pltpu.async_copy(x_hbm.at[pl.ds(r0, CHUNK)], rows_v.at[b], gsems[b])
            pltpu.async_copy(idx_hbm.at[pl.ds(r0, CHUNK)], idx_bufs[b], gsems[b])

        def wait_g(b):
            pltpu.make_async_copy(
                x_hbm.at[pl.ds(0, CHUNK)], rows_v.at[b], gsems[b]).wait()
            pltpu.make_async_copy(
                idx_hbm.at[pl.ds(0, CHUNK)], idx_bufs[b], gsems[b]).wait()

        def start_s(b):
            pltpu.async_copy(rows_v.at[b], acc_sh.at[idx_bufs[b]], ssems[b],
                             add=True)

        def wait_s(b):
            pltpu.make_async_copy(
                rows_v.at[b], acc_sh.at[idx_bufs[b]], ssems[b]).wait()

        for b in range(NBUF):
            start_g(b, b)

        for j in range(MAXC):
            b = j % NBUF

            def body(j=j, b=b):
                wait_g(b)
                start_s(b)
                if j >= 1:
                    wait_s((b - 1) % NBUF)
                    if j + NBUF - 1 < MAXC:
                        jn = j + NBUF - 1
                        pl.when(jn < count)(
                            lambda: start_g(jn, (b - 1) % NBUF))

            if j < BASE_PER_W:
                body()
            else:
                pl.when(j < count)(body)

        pl.when(count == MAXC)(lambda: wait_s((MAXC - 1) % NBUF))
        pl.when(count == MAXC - 1)(lambda: wait_s((MAXC - 2) % NBUF))
        plsc.subcore_barrier()

        pltpu.sync_copy(
            acc_sh.at[pl.ds(s * SEG_PER_TILE, SEG_PER_TILE)],
            out_hbm.at[c, pl.ds(s * SEG_PER_TILE, SEG_PER_TILE)],
        )

    return seg_sum(x, idx)


def _tc_head(partials, f2d, t2d, W1, b1_2d, W2, b2_2d):
    def head(p_ref, f_ref, t_ref, w1_ref, b1_ref, w2_ref, b2_ref, o_ref):
        xp = p_ref[0] + p_ref[1]
        h = jnp.dot(xp, w1_ref[pl.ds(0, D), :],
                    preferred_element_type=jnp.float32)
        h = h + f_ref[...] * w1_ref[pl.ds(D, 1), :]
        h = h + t_ref[...] * w1_ref[pl.ds(D + 1, 1), :]
        h = h + b1_ref[...]
        h = jnp.where(h >= 0.0, h, 0.01 * h)
        o_ref[...] = jnp.dot(h, w2_ref[...],
                             preferred_element_type=jnp.float32) + b2_ref[...]

    return pl.pallas_call(
        head,
        out_shape=jax.ShapeDtypeStruct((B, OUT_DIM), jnp.float32),
    )(partials, f2d, t2d, W1, b1_2d, W2, b2_2d)


def kernel(x, edge_index, batch, feature_index, threshold, W1, b1, W2, b2):
    partials = _sc_segment_sum(x, batch.astype(jnp.int32))
    out = _tc_head(
        partials,
        feature_index[:, None],
        threshold[:, None],
        W1,
        b1[None, :],
        W2,
        b2[None, :],
    )
    return out


# speedup vs baseline: 1.0312x; 1.0312x over previous
import functools

import jax
import jax.numpy as jnp
from jax import lax
from jax.experimental import pallas as pl
from jax.experimental.pallas import tpu as pltpu
from jax.experimental.pallas import tpu_sc as plsc

N_NODES = 100000
B = 2048
D = 128
HIDDEN = 64
OUT_DIM = 2

NC = 2
NS = 16
NW = NC * NS
CHUNK = 200
NCHUNKS = N_NODES // CHUNK
BASE_PER_W = NCHUNKS // NW
EXTRA = NCHUNKS - BASE_PER_W * NW
MAXC = BASE_PER_W + 1
NBUF = 4
SEG_PER_TILE = B // NS


def _sc_segment_sum(x, idx):
    mesh = plsc.VectorSubcoreMesh(core_axis_name="c", subcore_axis_name="s")

    @functools.partial(
        pl.kernel,
        mesh=mesh,
        out_type=jax.ShapeDtypeStruct((NC, B, D), jnp.float32),
        scratch_types=[
            pltpu.VMEM((NBUF, CHUNK, D), jnp.float32),
            pltpu.VMEM((CHUNK,), jnp.int32),
            pltpu.VMEM((CHUNK,), jnp.int32),
            pltpu.VMEM((CHUNK,), jnp.int32),
            pltpu.VMEM((CHUNK,), jnp.int32),
            pltpu.VMEM_SHARED((B, D), jnp.float32),
            pltpu.SemaphoreType.DMA,
            pltpu.SemaphoreType.DMA,
            pltpu.SemaphoreType.DMA,
            pltpu.SemaphoreType.DMA,
            pltpu.SemaphoreType.DMA,
            pltpu.SemaphoreType.DMA,
            pltpu.SemaphoreType.DMA,
            pltpu.SemaphoreType.DMA,
        ],
    )
    def seg_sum(x_hbm, idx_hbm, out_hbm, rows_v, idx_v0, idx_v1, idx_v2,
                idx_v3, acc_sh, g0, g1, g2, g3, s0, s1, s2, s3):
        c = lax.axis_index("c")
        s = lax.axis_index("s")
        wid = s * NC + c
        gsems = (g0, g1, g2, g3)
        ssems = (s0, s1, s2, s3)
        idx_bufs = (idx_v0, idx_v1, idx_v2, idx_v3)

        zvec = jnp.zeros((16,), jnp.float32)

        def zero_row(r, _):
            for j in range(D // 16):
                rows_v[0, r, pl.ds(j * 16, 16)] = zvec
            return 0

        lax.fori_loop(0, SEG_PER_TILE, zero_row, 0)
        pltpu.sync_copy(rows_v.at[0, pl.ds(0, SEG_PER_TILE)],
                        acc_sh.at[pl.ds(s * SEG_PER_TILE, SEG_PER_TILE)])
        plsc.subcore_barrier()

        first = BASE_PER_W * wid + jnp.minimum(wid, EXTRA)
        count = jnp.where(wid < EXTRA, BASE_PER_W + 1, BASE_PER_W)

        def start_g(j, b):
            r0 = (first + j) * CHUNK
            pltpu.async_copy(x_hbm.at[pl.ds(r0, CHUNK)], rows_v.at[b], gsems[b])
            pltpu.async_copy(idx_hbm.at[pl.ds(r0, CHUNK)], idx_bufs[b], gsems[b])

        def wait_g(b):
            pltpu.make_async_copy(
                x_hbm.at[pl.ds(0, CHUNK)], rows_v.at[b], gsems[b]).wait()
            pltpu.make_async_copy(
                idx_hbm.at[pl.ds(0, CHUNK)], idx_bufs[b], gsems[b]).wait()

        def start_s(b):
            pltpu.async_copy(rows_v.at[b], acc_sh.at[idx_bufs[b]], ssems[b],
                             add=True)

        def wait_s(b):
            pltpu.make_async_copy(
                rows_v.at[b], acc_sh.at[idx_bufs[b]], ssems[b]).wait()

        for b in range(NBUF):
            start_g(b, b)

        for j in range(MAXC):
            b = j % NBUF

            def body(j=j, b=b):
                wait_g(b)
                start_s(b)
                if j >= 1:
                    wait_s((b - 1) % NBUF)
                    if j + 3 < MAXC:
                        jn = j + 3
                        pl.when(jn < count)(
                            lambda: start_g(jn, (b - 1) % NBUF))

            if j < BASE_PER_W:
                body()
            else:
                pl.when(j < count)(body)

        pl.when(count == MAXC)(lambda: wait_s((MAXC - 1) % NBUF))
        pl.when(count == MAXC - 1)(lambda: wait_s((MAXC - 2) % NBUF))
        plsc.subcore_barrier()

        pltpu.sync_copy(
            acc_sh.at[pl.ds(s * SEG_PER_TILE, SEG_PER_TILE)],
            out_hbm.at[c, pl.ds(s * SEG_PER_TILE, SEG_PER_TILE)],
        )

    return seg_sum(x, idx)


def _tc_head(partials, f2d, t2d, W1, b1_2d, W2, b2_2d):
    def head(p_ref, f_ref, t_ref, w1_ref, b1_ref, w2_ref, b2_ref, o_ref):
        xp = p_ref[0] + p_ref[1]
        h = jnp.dot(xp, w1_ref[pl.ds(0, D), :],
                    preferred_element_type=jnp.float32)
        h = h + f_ref[...] * w1_ref[pl.ds(D, 1), :]
        h = h + t_ref[...] * w1_ref[pl.ds(D + 1, 1), :]
        h = h + b1_ref[...]
        h = jnp.where(h >= 0.0, h, 0.01 * h)
        o_ref[...] = jnp.dot(h, w2_ref[...],
                             preferred_element_type=jnp.float32) + b2_ref[...]

    return pl.pallas_call(
        head,
        out_shape=jax.ShapeDtypeStruct((B, 128), jnp.float32),
    )(partials, f2d, t2d, W1, b1_2d, W2, b2_2d)


def kernel(x, edge_index, batch, feature_index, threshold, W1, b1, W2, b2):
    partials = _sc_segment_sum(x, batch.astype(jnp.int32))
    W2p = jnp.pad(W2, ((0, 0), (0, 128 - OUT_DIM)))
    b2p = jnp.pad(b2, (0, 128 - OUT_DIM))
    out = _tc_head(
        partials,
        feature_index[:, None],
        threshold[:, None],
        W1,
        b1[None, :],
        W2p,
        b2p[None, :],
    )
    return out[:, :OUT_DIM]


# speedup vs baseline: 1.0475x; 1.0159x over previous
import functools

import jax
import jax.numpy as jnp
from jax import lax
from jax.experimental import pallas as pl
from jax.experimental.pallas import tpu as pltpu
from jax.experimental.pallas import tpu_sc as plsc

N_NODES = 100000
B = 2048
D = 128
HIDDEN = 64
OUT_DIM = 2

NC = 2
NS = 16
NW = NC * NS
CHUNK = 200
NCHUNKS = N_NODES // CHUNK
BASE_PER_W = NCHUNKS // NW
EXTRA = NCHUNKS - BASE_PER_W * NW
MAXC = BASE_PER_W + 1
NBUF = 4
SEG_PER_TILE = B // NS
ZROWS = 8


def _sc_segment_sum(x, idx):
    mesh = plsc.VectorSubcoreMesh(core_axis_name="c", subcore_axis_name="s")

    @functools.partial(
        pl.kernel,
        mesh=mesh,
        out_type=jax.ShapeDtypeStruct((NC, B, D), jnp.float32),
        scratch_types=[
            pltpu.VMEM((NBUF, CHUNK, D), jnp.float32),
            pltpu.VMEM((CHUNK,), jnp.int32),
            pltpu.VMEM((CHUNK,), jnp.int32),
            pltpu.VMEM((CHUNK,), jnp.int32),
            pltpu.VMEM((CHUNK,), jnp.int32),
            pltpu.VMEM((ZROWS, D), jnp.float32),
            pltpu.VMEM_SHARED((B, D), jnp.float32),
            pltpu.SemaphoreType.DMA,
            pltpu.SemaphoreType.DMA,
            pltpu.SemaphoreType.DMA,
            pltpu.SemaphoreType.DMA,
            pltpu.SemaphoreType.DMA,
            pltpu.SemaphoreType.DMA,
            pltpu.SemaphoreType.DMA,
            pltpu.SemaphoreType.DMA,
        ],
    )
    def seg_sum(x_hbm, idx_hbm, out_hbm, rows_v, idx_v0, idx_v1, idx_v2,
                idx_v3, zero_v, acc_sh, g0, g1, g2, g3, s0, s1, s2, s3):
        c = lax.axis_index("c")
        s = lax.axis_index("s")
        wid = s * NC + c
        gsems = (g0, g1, g2, g3)
        ssems = (s0, s1, s2, s3)
        idx_bufs = (idx_v0, idx_v1, idx_v2, idx_v3)

        first = BASE_PER_W * wid + jnp.minimum(wid, EXTRA)
        count = jnp.where(wid < EXTRA, BASE_PER_W + 1, BASE_PER_W)

        def start_g(j, b):
            r0 = (first + j) * CHUNK
            pltpu.async_copy(x_hbm.at[pl.ds(r0, CHUNK)], rows_v.at[b], gsems[b])
            pltpu.async_copy(idx_hbm.at[pl.ds(r0, CHUNK)], idx_bufs[b], gsems[b])

        def wait_g(b):
            pltpu.make_async_copy(
                x_hbm.at[pl.ds(0, CHUNK)], rows_v.at[b], gsems[b]).wait()
            pltpu.make_async_copy(
                idx_hbm.at[pl.ds(0, CHUNK)], idx_bufs[b], gsems[b]).wait()

        def start_s(b):
            pltpu.async_copy(rows_v.at[b], acc_sh.at[idx_bufs[b]], ssems[b],
                             add=True)

        def wait_s(b):
            pltpu.make_async_copy(
                rows_v.at[b], acc_sh.at[idx_bufs[b]], ssems[b]).wait()

        for b in range(NBUF):
            start_g(b, b)

        zvec = jnp.zeros((16,), jnp.float32)

        def zero_row(r, _):
            for j in range(D // 16):
                zero_v[r, pl.ds(j * 16, 16)] = zvec
            return 0

        lax.fori_loop(0, ZROWS, zero_row, 0)

        def zero_slice(i, _):
            pltpu.sync_copy(
                zero_v, acc_sh.at[pl.ds(s * SEG_PER_TILE + i * ZROWS, ZROWS)])
            return 0

        lax.fori_loop(0, SEG_PER_TILE // ZROWS, zero_slice, 0)
        plsc.subcore_barrier()

        def group(gi, _):
            for u in range(NBUF):
                j = NBUF * gi + u

                def sub(u=u, j=j):
                    wait_g(u)
                    start_s(u)

                    def after_first():
                        wait_s((u - 1) % NBUF)
                        jn = j + NBUF - 1
                        pl.when(jn < count)(
                            lambda: start_g(jn, (u - 1) % NBUF))

                    pl.when(j >= 1)(after_first)

                pl.when(j < count)(sub)
            return 0

        lax.fori_loop(0, (MAXC + NBUF - 1) // NBUF, group, 0)

        pl.when(count == MAXC)(lambda: wait_s((MAXC - 1) % NBUF))
        pl.when(count == MAXC - 1)(lambda: wait_s((MAXC - 2) % NBUF))
        plsc.subcore_barrier()

        pltpu.sync_copy(
            acc_sh.at[pl.ds(s * SEG_PER_TILE, SEG_PER_TILE)],
            out_hbm.at[c, pl.ds(s * SEG_PER_TILE, SEG_PER_TILE)],
        )

    return seg_sum(x, idx)


def _tc_head(partials, f2d, t2d, W1, b1_2d, W2, b2_2d):
    def head(p_ref, f_ref, t_ref, w1_ref, b1_ref, w2_ref, b2_ref, o_ref):
        xp = p_ref[0] + p_ref[1]
        h = jnp.dot(xp, w1_ref[pl.ds(0, D), :],
                    preferred_element_type=jnp.float32)
        h = h + f_ref[...] * w1_ref[pl.ds(D, 1), :]
        h = h + t_ref[...] * w1_ref[pl.ds(D + 1, 1), :]
        h = h + b1_ref[...]
        h = jnp.where(h >= 0.0, h, 0.01 * h)
        o_ref[...] = jnp.dot(h, w2_ref[...],
                             preferred_element_type=jnp.float32) + b2_ref[...]

    return pl.pallas_call(
        head,
        out_shape=jax.ShapeDtypeStruct((B, 128), jnp.float32),
    )(partials, f2d, t2d, W1, b1_2d, W2, b2_2d)


def kernel(x, edge_index, batch, feature_index, threshold, W1, b1, W2, b2):
    partials = _sc_segment_sum(x, batch.astype(jnp.int32))
    W2p = jnp.pad(W2, ((0, 0), (0, 128 - OUT_DIM)))
    b2p = jnp.pad(b2, (0, 128 - OUT_DIM))
    out = _tc_head(
        partials,
        feature_index[:, None],
        threshold[:, None],
        W1,
        b1[None, :],
        W2p,
        b2p[None, :],
    )
    return out[:, :OUT_DIM]
